# baseline (speedup 1.0000x reference)
import jax
import jax.numpy as jnp
from jax import lax
from jax.experimental import pallas as pl
from jax.experimental.pallas import tpu as pltpu
from jax.experimental.pallas import tpu_sc as plsc

_N_ROWS = 4096
_N_COLS = 100000
_NC = 2
_NS = 16
_L = 16
_NW = _NC * _NS
_ROWS_PER_TILE = _N_ROWS // _NW
_CHUNKS = _ROWS_PER_TILE // _L


def _ganloss_body(pf_hbm, tgt_hbm, rew_hbm, out_hbm,
                  tgt_v, idx_v, rew_v, vals_v, buf_v, sem, sem2):
    cid = lax.axis_index("c")
    sid = lax.axis_index("s")
    wid = sid * _NC + cid
    base = wid * _ROWS_PER_TILE

    tgt_cp = pltpu.make_async_copy(
        tgt_hbm.at[pl.ds(base, _ROWS_PER_TILE)], tgt_v, sem)
    tgt_cp.start()
    rew_cp = pltpu.make_async_copy(
        rew_hbm.at[pl.ds(base, _ROWS_PER_TILE)], rew_v, sem2)
    rew_cp.start()
    tgt_cp.wait()

    lanes = lax.iota(jnp.int32, _L)
    for j0 in range(_CHUNKS):
        c = tgt_v[pl.ds(j0 * _L, _L)]
        i = (base + j0 * _L) + lanes
        ih = lax.shift_right_logical(i, 7)
        il = jnp.bitwise_and(i, jnp.int32(127))
        ch = lax.shift_right_logical(c, 3)
        cl = jnp.bitwise_and(c, jnp.int32(7))
        idx_v[pl.ds(j0 * _L, _L)] = (ch * 32 + ih) * 1024 + cl * 128 + il

    cp = pltpu.make_async_copy(pf_hbm.at[idx_v], vals_v, sem)
    cp.start()
    rew_cp.wait()
    cp.wait()

    acc = jnp.zeros((_L,), jnp.float32)
    for j0 in range(_CHUNKS):
        acc = acc + vals_v[pl.ds(j0 * _L, _L)] * rew_v[pl.ds(j0 * _L, _L)]

    buf_v[...] = -acc
    pltpu.sync_copy(buf_v, out_hbm.at[pl.ds(wid * _L, _L)])


@jax.jit
def _ganloss(prob, target_i32, reward):
    pf = jnp.transpose(
        prob.reshape(32, 128, 12500, 8), (2, 0, 3, 1)).reshape(-1)
    mesh = plsc.VectorSubcoreMesh(core_axis_name="c", subcore_axis_name="s")
    run = pl.kernel(
        _ganloss_body,
        out_type=jax.ShapeDtypeStruct((_NW * _L,), jnp.float32),
        mesh=mesh,
        compiler_params=pltpu.CompilerParams(needs_layout_passes=False),
        scratch_types=[
            pltpu.VMEM((_ROWS_PER_TILE,), jnp.int32),
            pltpu.VMEM((_ROWS_PER_TILE,), jnp.int32),
            pltpu.VMEM((_ROWS_PER_TILE,), jnp.float32),
            pltpu.VMEM((_ROWS_PER_TILE,), jnp.float32),
            pltpu.VMEM((_L,), jnp.float32),
            pltpu.SemaphoreType.DMA,
            pltpu.SemaphoreType.DMA,
        ],
    )
    out = run(pf, target_i32, reward)
    return jnp.sum(out)


def kernel(prob, target, reward):
    return _ganloss(prob, target.astype(jnp.int32), reward)
